# baseline (speedup 1.0000x reference)
import functools

import jax
import jax.numpy as jnp
from jax import lax
from jax.experimental import pallas as pl
from jax.experimental.pallas import tpu as pltpu
from jax.experimental.pallas import tpu_sc as plsc

N_NODES = 10000
R = 16
K = 4
C = 128
LANES = 16
_SEGS = ((0, 1), (1, 3), (4, 5), (9, 7))
_SCALES = (1.0, 1.0 / 3.0, 1.0 / 5.0, 1.0 / 7.0)

TN = 610
N_TC = 4880
N_SC = N_NODES - N_TC

NB = 16
NBLK = N_SC // NB
NC = 2
NS = 16
NW = NC * NS


def _compute_block(in_v, out_v):
    def n_body(n, carry):
        for cg in range(C // LANES):
            sl = pl.ds(cg * LANES, LANES)
            sq = [None] * R
            for r in range(R):
                x = in_v[n, r, sl]
                sq[r] = x * x
            s1 = (sq[1] + sq[2]) + sq[3]
            s2 = ((sq[4] + sq[5]) + (sq[6] + sq[7])) + sq[8]
            s3 = (((sq[9] + sq[10]) + (sq[11] + sq[12]))
                  + ((sq[13] + sq[14]) + sq[15]))
            out_v[n, 0, sl] = sq[0]
            out_v[n, 1, sl] = s1 * _SCALES[1]
            out_v[n, 2, sl] = s2 * _SCALES[2]
            out_v[n, 3, sl] = s3 * _SCALES[3]
        return carry

    lax.fori_loop(0, NB, n_body, 0)


def _sc_body(x_hbm, out_hbm, in_a, in_b, out_a, out_b,
             in_sem_a, in_sem_b, out_sem_a, out_sem_b):
    wid = lax.axis_index("s") * NC + lax.axis_index("c")
    n_mine = (NBLK - wid + NW - 1) // NW
    pairs = n_mine // 2

    def in_base(i):
        return N_TC + (wid + i * NW) * NB

    def out_base(i):
        return (wid + i * NW) * NB

    def wait_in(buf, sem):
        pltpu.make_async_copy(x_hbm.at[pl.ds(0, NB)], buf, sem).wait()

    def wait_out(buf, sem):
        pltpu.make_async_copy(buf, out_hbm.at[pl.ds(0, NB)], sem).wait()

    pltpu.async_copy(x_hbm.at[pl.ds(in_base(0), NB)], in_a, in_sem_a)
    pltpu.async_copy(x_hbm.at[pl.ds(in_base(1), NB)], in_b, in_sem_b)

    def half(j, i, in_v, in_sem, out_v, out_sem):
        wait_in(in_v, in_sem)

        @pl.when(j > 0)
        def _():
            wait_out(out_v, out_sem)

        _compute_block(in_v, out_v)
        pltpu.async_copy(out_v, out_hbm.at[pl.ds(out_base(i), NB)], out_sem)

        @pl.when(i + 2 < n_mine)
        def _():
            pltpu.async_copy(x_hbm.at[pl.ds(in_base(i + 2), NB)], in_v, in_sem)

    def pair_body(j, carry):
        half(j, 2 * j, in_a, in_sem_a, out_a, out_sem_a)
        half(j, 2 * j + 1, in_b, in_sem_b, out_b, out_sem_b)
        return carry

    lax.fori_loop(0, pairs, pair_body, 0)

    wait_out(out_a, out_sem_a)
    wait_out(out_b, out_sem_b)


def _sc_call(input_tensor):
    mesh = plsc.VectorSubcoreMesh(core_axis_name="c", subcore_axis_name="s")
    f = functools.partial(
        pl.kernel,
        out_type=jax.ShapeDtypeStruct((N_SC, K, C), jnp.float32),
        mesh=mesh,
        scratch_types=[
            pltpu.VMEM((NB, R, C), jnp.float32),
            pltpu.VMEM((NB, R, C), jnp.float32),
            pltpu.VMEM((NB, K, C), jnp.float32),
            pltpu.VMEM((NB, K, C), jnp.float32),
            pltpu.SemaphoreType.DMA,
            pltpu.SemaphoreType.DMA,
            pltpu.SemaphoreType.DMA,
            pltpu.SemaphoreType.DMA,
        ],
    )(_sc_body)
    return f(input_tensor)


def _tc_body(x_ref, o_ref):
    x = x_ref[...]
    sq = x * x
    for k, (r0, d) in enumerate(_SEGS):
        acc = sq[:, r0, :]
        for r in range(r0 + 1, r0 + d):
            acc = acc + sq[:, r, :]
        o_ref[:, k, :] = acc * _SCALES[k]


def _tc_call(input_tensor):
    return pl.pallas_call(
        _tc_body,
        grid=(N_TC // TN,),
        in_specs=[pl.BlockSpec((TN, R, C), lambda i: (i, 0, 0))],
        out_specs=pl.BlockSpec((TN, K, C), lambda i: (i, 0, 0)),
        out_shape=jax.ShapeDtypeStruct((N_NODES, K, C), jnp.float32),
    )(input_tensor)


@jax.jit
def kernel(input_tensor):
    sc_out = _sc_call(input_tensor)
    tc_out = _tc_call(input_tensor)
    return lax.dynamic_update_slice(tc_out, sc_out, (N_TC, 0, 0))


# speedup vs baseline: 1.0640x; 1.0640x over previous
import functools

import jax
import jax.numpy as jnp
from jax import lax
from jax.experimental import pallas as pl
from jax.experimental.pallas import tpu as pltpu
from jax.experimental.pallas import tpu_sc as plsc

N_NODES = 10000
R = 16
K = 4
C = 128
LANES = 16
_SEGS = ((0, 1), (1, 3), (4, 5), (9, 7))
_SCALES = (1.0, 1.0 / 3.0, 1.0 / 5.0, 1.0 / 7.0)

TN = 984
N_TC = 5904
N_SC = N_NODES - N_TC

NB = 16
NBLK = N_SC // NB
NC = 2
NS = 16
NW = NC * NS


def _compute_block(in_v, out_v):
    def n_body(n, carry):
        for cg in range(C // LANES):
            sl = pl.ds(cg * LANES, LANES)
            sq = [None] * R
            for r in range(R):
                x = in_v[n, r, sl]
                sq[r] = x * x
            s1 = (sq[1] + sq[2]) + sq[3]
            s2 = ((sq[4] + sq[5]) + (sq[6] + sq[7])) + sq[8]
            s3 = (((sq[9] + sq[10]) + (sq[11] + sq[12]))
                  + ((sq[13] + sq[14]) + sq[15]))
            out_v[n, 0, sl] = sq[0]
            out_v[n, 1, sl] = s1 * _SCALES[1]
            out_v[n, 2, sl] = s2 * _SCALES[2]
            out_v[n, 3, sl] = s3 * _SCALES[3]
        return carry

    lax.fori_loop(0, NB, n_body, 0)


def _sc_body(x_hbm, out_hbm, in_a, in_b, out_a, out_b,
             in_sem_a, in_sem_b, out_sem_a, out_sem_b):
    wid = lax.axis_index("s") * NC + lax.axis_index("c")
    n_mine = (NBLK - wid + NW - 1) // NW
    pairs = n_mine // 2

    def in_base(i):
        return N_TC + (wid + i * NW) * NB

    def out_base(i):
        return (wid + i * NW) * NB

    def wait_in(buf, sem):
        pltpu.make_async_copy(x_hbm.at[pl.ds(0, NB)], buf, sem).wait()

    def wait_out(buf, sem):
        pltpu.make_async_copy(buf, out_hbm.at[pl.ds(0, NB)], sem).wait()

    pltpu.async_copy(x_hbm.at[pl.ds(in_base(0), NB)], in_a, in_sem_a)
    pltpu.async_copy(x_hbm.at[pl.ds(in_base(1), NB)], in_b, in_sem_b)

    def half(j, i, in_v, in_sem, out_v, out_sem):
        wait_in(in_v, in_sem)

        @pl.when(j > 0)
        def _():
            wait_out(out_v, out_sem)

        _compute_block(in_v, out_v)
        pltpu.async_copy(out_v, out_hbm.at[pl.ds(out_base(i), NB)], out_sem)

        @pl.when(i + 2 < n_mine)
        def _():
            pltpu.async_copy(x_hbm.at[pl.ds(in_base(i + 2), NB)], in_v, in_sem)

    def pair_body(j, carry):
        half(j, 2 * j, in_a, in_sem_a, out_a, out_sem_a)
        half(j, 2 * j + 1, in_b, in_sem_b, out_b, out_sem_b)
        return carry

    lax.fori_loop(0, pairs, pair_body, 0)

    wait_out(out_a, out_sem_a)
    wait_out(out_b, out_sem_b)


def _sc_call(input_tensor):
    mesh = plsc.VectorSubcoreMesh(core_axis_name="c", subcore_axis_name="s")
    f = functools.partial(
        pl.kernel,
        out_type=jax.ShapeDtypeStruct((N_SC, K, C), jnp.float32),
        mesh=mesh,
        scratch_types=[
            pltpu.VMEM((NB, R, C), jnp.float32),
            pltpu.VMEM((NB, R, C), jnp.float32),
            pltpu.VMEM((NB, K, C), jnp.float32),
            pltpu.VMEM((NB, K, C), jnp.float32),
            pltpu.SemaphoreType.DMA,
            pltpu.SemaphoreType.DMA,
            pltpu.SemaphoreType.DMA,
            pltpu.SemaphoreType.DMA,
        ],
    )(_sc_body)
    return f(input_tensor)


def _tc_body(x_ref, o_ref):
    x = x_ref[...]
    sq = x * x
    for k, (r0, d) in enumerate(_SEGS):
        acc = sq[:, r0, :]
        for r in range(r0 + 1, r0 + d):
            acc = acc + sq[:, r, :]
        o_ref[:, k, :] = acc * _SCALES[k]


def _tc_call(input_tensor):
    return pl.pallas_call(
        _tc_body,
        grid=(N_TC // TN,),
        in_specs=[pl.BlockSpec((TN, R, C), lambda i: (i, 0, 0))],
        out_specs=pl.BlockSpec((TN, K, C), lambda i: (i, 0, 0)),
        out_shape=jax.ShapeDtypeStruct((N_NODES, K, C), jnp.float32),
    )(input_tensor)


@jax.jit
def kernel(input_tensor):
    sc_out = _sc_call(input_tensor)
    tc_out = _tc_call(input_tensor)
    return lax.dynamic_update_slice(tc_out, sc_out, (N_TC, 0, 0))
